# baseline (speedup 1.0000x reference)
import functools

import jax
import jax.numpy as jnp
from jax import lax
from jax.experimental import pallas as pl
from jax.experimental.pallas import tpu as pltpu
from jax.experimental.pallas import tpu_sc as plsc

NUM_CHANNEL = 8
VOCAB = 100000
HIDDEN = 128
MBS = 4
SEQ = 2048

NPOS = SEQ * MBS
LANES = 16

_info = plsc.get_sparse_core_info()
NC = _info.num_cores
NS = _info.num_subcores
NW = NC * NS
PPW = NPOS // NW
CHUNK = 32
NCHUNK = PPW // CHUNK
NBUF = 3

_mesh = plsc.VectorSubcoreMesh(core_axis_name="c", subcore_axis_name="s")


@functools.partial(
    pl.kernel,
    mesh=_mesh,
    out_type=jax.ShapeDtypeStruct((NPOS, HIDDEN), jnp.float32),
    scratch_types=[
        pltpu.VMEM((NUM_CHANNEL, PPW // 128, 128), jnp.int32),
        pltpu.VMEM((NBUF, NUM_CHANNEL, CHUNK, HIDDEN), jnp.float32),
        pltpu.VMEM((2, CHUNK, HIDDEN), jnp.float32),
        pltpu.SemaphoreType.DMA,
        pltpu.SemaphoreType.DMA,
        pltpu.SemaphoreType.DMA,
        pltpu.SemaphoreType.DMA,
        pltpu.SemaphoreType.DMA,
        pltpu.SemaphoreType.DMA,
    ],
)
def _sc_embed(ids_hbm, tab_hbm, out_hbm, ids_v, gbuf, obuf,
              isem, g0, g1, g2, o0, o1):
    wid = lax.axis_index("s") * NC + lax.axis_index("c")
    gsem = (g0, g1, g2)
    osem = (o0, o1)

    icopies = [
        pltpu.async_copy(
            ids_hbm.at[c, pl.ds(wid * (PPW // 128), PPW // 128)],
            ids_v.at[c], isem,
        )
        for c in range(NUM_CHANNEL)
    ]
    for cp in icopies:
        cp.wait()

    def fire(k, j):
        r, col = (k * CHUNK) // 128, (k * CHUNK) % 128
        return [
            pltpu.async_copy(
                tab_hbm.at[c].at[ids_v.at[c, r, pl.ds(col, CHUNK)]],
                gbuf.at[j, c],
                gsem[j],
            )
            for c in range(NUM_CHANNEL)
        ]

    gcopies = [fire(0, 0), fire(1, 1), None]
    scopies = [None, None]

    for k in range(NCHUNK):
        j = k % NBUF
        jo = k % 2
        for cp in gcopies[j]:
            cp.wait()
        if k + 2 < NCHUNK:
            gcopies[(k + 2) % NBUF] = fire(k + 2, (k + 2) % NBUF)
        if scopies[jo] is not None:
            scopies[jo].wait()
            scopies[jo] = None

        def pos_body(p, carry, _j=j, _jo=jo):
            for h in range(HIDDEN // LANES):
                sl = pl.ds(h * LANES, LANES)
                acc = gbuf[_j, 0, p, sl]
                for c in range(1, NUM_CHANNEL):
                    acc = acc + gbuf[_j, c, p, sl]
                obuf[_jo, p, sl] = acc
            return carry

        lax.fori_loop(0, CHUNK, pos_body, 0, unroll=False)

        base = wid * PPW + k * CHUNK
        scopies[jo] = pltpu.async_copy(
            obuf.at[jo], out_hbm.at[pl.ds(base, CHUNK)], osem[jo]
        )

    for cp in scopies:
        if cp is not None:
            cp.wait()


def kernel(audio_ids, tables):
    ids_t = jnp.transpose(audio_ids, (2, 1, 0)).reshape(
        NUM_CHANNEL, NPOS // 128, 128
    )
    out = _sc_embed(ids_t, tables)
    return out.reshape(SEQ, MBS, HIDDEN)
